# Initial kernel scaffold; baseline (speedup 1.0000x reference)
import jax
import jax.numpy as jnp
from jax.experimental import pallas as pl


def kernel(tokens, table):
    raise NotImplementedError("write your pallas kernel here")


# speedup vs baseline: 1.4011x; 1.4011x over previous
import functools
import math

import jax
import jax.numpy as jnp
from jax import lax
from jax.experimental import pallas as pl
from jax.experimental.pallas import tpu as pltpu
from jax.experimental.pallas import tpu_sc as plsc

D = 32
SCALE = math.sqrt(32.0)
NC, NS = 2, 16
NW = NC * NS
B = 4096 * 200
BPW = B // NW
SUB = 128
C = 1024
NSUB = C // SUB
NCHUNK = BPW // C
TOK_ROWS_PER_W = BPW // SUB

_mesh = plsc.VectorSubcoreMesh(core_axis_name="c", subcore_axis_name="s")


@functools.partial(
    pl.kernel,
    mesh=_mesh,
    out_type=jax.ShapeDtypeStruct((B, D), jnp.float32),
    compiler_params=pltpu.CompilerParams(use_tc_tiling_on_sc=False),
    scratch_types=[
        pltpu.VMEM((NSUB, SUB), jnp.int32),
        pltpu.VMEM((C, D), jnp.float32),
        pltpu.SemaphoreType.DMA,
    ],
)
def _embed_sc(tok_hbm, table_hbm, out_hbm, idx_v, rows_v, gsem):
    wid = lax.axis_index("s") * NC + lax.axis_index("c")
    tok_row0 = wid * TOK_ROWS_PER_W
    out_row0 = wid * BPW

    def chunk_body(g, _):
        pltpu.sync_copy(tok_hbm.at[pl.ds(tok_row0 + g * NSUB, NSUB)], idx_v)
        copies = []
        for j in range(NSUB):
            copies.append(
                pltpu.make_async_copy(
                    table_hbm.at[idx_v.at[j]],
                    rows_v.at[pl.ds(j * SUB, SUB)],
                    gsem,
                )
            )
        for c in copies:
            c.start()
        for c in copies:
            c.wait()

        def scale_body(i, _):
            r0 = i * 8
            for rr in range(8):
                for h in (0, 16):
                    rows_v[r0 + rr, pl.ds(h, 16)] = (
                        rows_v[r0 + rr, pl.ds(h, 16)] * SCALE
                    )
            return 0

        lax.fori_loop(0, C // 8, scale_body, 0)

        pltpu.sync_copy(rows_v, out_hbm.at[pl.ds(out_row0 + g * C, C)])
        return 0

    lax.fori_loop(0, NCHUNK, chunk_body, 0)


def kernel(tokens, table):
    tok = tokens.astype(jnp.int32).reshape(B // SUB, SUB)
    out = _embed_sc(tok, table)
    return out.reshape(tokens.shape[0], tokens.shape[1], D)


# speedup vs baseline: 1.4772x; 1.0544x over previous
import functools
import math

import jax
import jax.numpy as jnp
from jax import lax
from jax.experimental import pallas as pl
from jax.experimental.pallas import tpu as pltpu
from jax.experimental.pallas import tpu_sc as plsc

D = 32
SCALE = math.sqrt(32.0)
NC, NS = 2, 16
NW = NC * NS
B = 4096 * 200
BPW = B // NW
SUB = 128
C = 1280
NSUB = C // SUB
NCHUNK = BPW // C
TOK_ROWS_PER_W = BPW // SUB

_mesh = plsc.VectorSubcoreMesh(core_axis_name="c", subcore_axis_name="s")


@functools.partial(
    pl.kernel,
    mesh=_mesh,
    out_type=jax.ShapeDtypeStruct((B, D), jnp.float32),
    compiler_params=pltpu.CompilerParams(use_tc_tiling_on_sc=False),
    scratch_types=[
        pltpu.VMEM((TOK_ROWS_PER_W, SUB), jnp.int32),
        pltpu.VMEM((2 * C, D), jnp.float32),
        pltpu.SemaphoreType.DMA,
        pltpu.SemaphoreType.DMA,
    ],
)
def _embed_sc(tok_hbm, table_hbm, out_hbm, idx_v, rows_v, gsem0, gsem1):
    wid = lax.axis_index("s") * NC + lax.axis_index("c")
    tok_row0 = wid * TOK_ROWS_PER_W
    out_row0 = wid * BPW
    sems = (gsem0, gsem1)

    pltpu.sync_copy(tok_hbm.at[pl.ds(tok_row0, TOK_ROWS_PER_W)], idx_v)

    def fire(g, b):
        for j in range(NSUB):
            pltpu.make_async_copy(
                table_hbm.at[idx_v.at[g * NSUB + j]],
                rows_v.at[pl.ds(b * C + j * SUB, SUB)],
                sems[b],
            ).start()

    def drain(b):
        pltpu.make_async_copy(
            out_hbm.at[pl.ds(0, C)],
            rows_v.at[pl.ds(b * C, C)],
            sems[b],
        ).wait()

    def scale(b):
        def scale_body(i, _):
            r0 = b * C + i * 8
            for rr in range(8):
                for h in (0, 16):
                    rows_v[r0 + rr, pl.ds(h, 16)] = (
                        rows_v[r0 + rr, pl.ds(h, 16)] * SCALE
                    )
            return 0

        lax.fori_loop(0, C // 8, scale_body, 0)

    def write(g, b):
        pltpu.sync_copy(
            rows_v.at[pl.ds(b * C, C)],
            out_hbm.at[pl.ds(out_row0 + g * C, C)],
        )

    fire(0, 0)

    def body(i, _):
        g0 = 2 * i
        fire(g0 + 1, 1)
        drain(0)
        scale(0)
        write(g0, 0)
        fire(g0 + 2, 0)
        drain(1)
        scale(1)
        write(g0 + 1, 1)
        return 0

    lax.fori_loop(0, NCHUNK // 2 - 1, body, 0)

    fire(NCHUNK - 1, 1)
    drain(0)
    scale(0)
    write(NCHUNK - 2, 0)
    drain(1)
    scale(1)
    write(NCHUNK - 1, 1)


def kernel(tokens, table):
    tok = tokens.astype(jnp.int32).reshape(B // SUB, SUB)
    out = _embed_sc(tok, table)
    return out.reshape(tokens.shape[0], tokens.shape[1], D)
